# baseline (speedup 1.0000x reference)
import functools

import jax
import jax.numpy as jnp
from jax import lax
from jax.experimental import pallas as pl
from jax.experimental.pallas import tpu as pltpu
from jax.experimental.pallas import tpu_sc as plsc

N_PAD = 10240
NNZ_PAD = 327680
NNZ_T = 10240
NCHUNK = 80
CHUNK_R = 128
FH = 64
F = 128
BLK = 128
PAD_IDX = 10016
NW = 32

_MESH = plsc.VectorSubcoreMesh(core_axis_name="c", subcore_axis_name="s",
                               num_cores=2, num_subcores=16)


def _pre_body(x_ref, ha_ref, W_ref, attn_ref, atte_ref, temb_ref, Wt_ref, bt_ref, b_ref,
              xt_ref, he_ref, an_ref, ae_ref, tvec_ref, shift_ref, mx_ref):
    i = pl.program_id(0)
    W = W_ref[...]
    xt = jnp.dot(x_ref[...], W, preferred_element_type=jnp.float32)
    he = jnp.dot(ha_ref[...], W, preferred_element_type=jnp.float32)
    xt_ref[...] = xt
    he_ref[...] = he
    an = jnp.sum(xt * attn_ref[...], axis=1)
    ae = jnp.sum(he * atte_ref[...], axis=1)
    an_ref[...] = an[None, None, :]
    ae_ref[...] = ae[None, None, :]

    @pl.when(i == 0)
    def _init():
        mx_ref[0] = -jnp.inf
        mx_ref[1] = -jnp.inf
        t = jax.nn.silu(temb_ref[...])
        tvec = jnp.dot(t, Wt_ref[...], preferred_element_type=jnp.float32)
        tvec_ref[...] = tvec + bt_ref[...][None, :] + b_ref[...][None, :]

    mx_ref[0] = jnp.maximum(mx_ref[0], jnp.max(an))
    mx_ref[1] = jnp.maximum(mx_ref[1], jnp.max(ae))

    @pl.when(i == pl.num_programs(0) - 1)
    def _fin():
        m = mx_ref[0] + mx_ref[1]
        shift_ref[0] = jnp.where(m > 0, m, 0.2 * m)


def _preamble(x_p, ha_p, W, attn, atte, temb, Wt, bt, b):
    nblk = N_PAD // BLK
    full = lambda i: (0, 0)
    return pl.pallas_call(
        _pre_body,
        grid=(nblk,),
        in_specs=[
            pl.BlockSpec((BLK, F), lambda i: (i, 0)),
            pl.BlockSpec((BLK, F), lambda i: (i, 0)),
            pl.BlockSpec((F, F), full),
            pl.BlockSpec((1, F), full),
            pl.BlockSpec((1, F), full),
            pl.BlockSpec((1, 512), full),
            pl.BlockSpec((512, F), full),
            pl.BlockSpec((F,), lambda i: (0,)),
            pl.BlockSpec((F,), lambda i: (0,)),
        ],
        out_specs=[
            pl.BlockSpec((BLK, F), lambda i: (i, 0)),
            pl.BlockSpec((BLK, F), lambda i: (i, 0)),
            pl.BlockSpec((1, 1, BLK), lambda i: (i, 0, 0)),
            pl.BlockSpec((1, 1, BLK), lambda i: (i, 0, 0)),
            pl.BlockSpec((1, F), full),
            pl.BlockSpec(memory_space=pltpu.SMEM),
        ],
        out_shape=[
            jax.ShapeDtypeStruct((N_PAD, F), jnp.float32),
            jax.ShapeDtypeStruct((N_PAD, F), jnp.float32),
            jax.ShapeDtypeStruct((nblk, 1, BLK), jnp.float32),
            jax.ShapeDtypeStruct((nblk, 1, BLK), jnp.float32),
            jax.ShapeDtypeStruct((1, F), jnp.float32),
            jax.ShapeDtypeStruct((1,), jnp.float32),
        ],
        scratch_shapes=[pltpu.SMEM((2,), jnp.float32)],
    )(x_p, ha_p, W, attn, atte, temb, Wt, bt, b)


def _sc_logits_body(nid_hbm, eid_hbm, an_hbm, ae_hbm, shift_hbm, riota_hbm,
                    ex_hbm, part_hbm,
                    nid_v, eid_v, ex_v, an_t, ae_t, sh16, riota_v,
                    acc_es, acc_d, acc_b, zbuf,
                    sh_es, sh_d, sh_b):
    c = lax.axis_index("c")
    s = lax.axis_index("s")
    w = c * 16 + s
    pltpu.sync_copy(an_hbm, an_t)
    pltpu.sync_copy(ae_hbm, ae_t)
    pltpu.sync_copy(shift_hbm, sh16)
    pltpu.sync_copy(riota_hbm, riota_v)
    pltpu.sync_copy(nid_hbm.at[w], nid_v)
    pltpu.sync_copy(eid_hbm.at[w], eid_v)
    shv = sh16[...]
    z16 = jnp.zeros((16,), jnp.float32)

    def zero_body(i, _):
        acc_es[i, :] = z16
        acc_d[i, :] = z16
        acc_b[i, :] = z16
        return 0
    lax.fori_loop(0, N_PAD // 16, zero_body, 0)

    def zb_body(i, _):
        zbuf[i, :] = z16
        return 0
    lax.fori_loop(0, 40, zb_body, 0)

    ones16 = jnp.ones((16,), jnp.float32)

    def body(i, _):
        sl = pl.ds(i * 16, 16)
        ni = nid_v[sl]
        ei = eid_v[sl]
        a = plsc.load_gather(an_t, [ni]) + plsc.load_gather(ae_t, [ei])
        a = jnp.where(a > 0, a, 0.2 * a)
        ex = jnp.exp(a - shv)
        ex_v[sl] = ex
        nhi, nlo = ni // 16, ni % 16
        plsc.addupdate_scatter(acc_es, [nhi, nlo], ex)
        plsc.addupdate_scatter(acc_d, [nhi, nlo], ones16)
        plsc.addupdate_scatter(acc_b, [ei // 16, ei % 16], ones16)
        return 0
    lax.fori_loop(0, NNZ_T // 16, body, 0)
    pltpu.sync_copy(ex_v, ex_hbm.at[w])

    rsl = pl.ds(s * 40, 40)
    pltpu.sync_copy(zbuf, sh_es.at[rsl])
    pltpu.sync_copy(zbuf, sh_d.at[rsl])
    pltpu.sync_copy(zbuf, sh_b.at[rsl])
    plsc.subcore_barrier()
    for j in range(5):
        src = pl.ds(j * 128, 128)
        idx = riota_v.at[j]
        pltpu.sync_copy(acc_es.at[src], sh_es.at[idx], add=True)
        pltpu.sync_copy(acc_d.at[src], sh_d.at[idx], add=True)
        pltpu.sync_copy(acc_b.at[src], sh_b.at[idx], add=True)
    plsc.subcore_barrier()
    pltpu.sync_copy(sh_es.at[rsl], part_hbm.at[c, 0, rsl])
    pltpu.sync_copy(sh_d.at[rsl], part_hbm.at[c, 1, rsl])
    pltpu.sync_copy(sh_b.at[rsl], part_hbm.at[c, 2, rsl])


def _sc_logits(nid2, eid2, a_n, a_e, shift16, riota):
    kfn = pl.kernel(
        _sc_logits_body,
        out_type=[
            jax.ShapeDtypeStruct((NW, NNZ_T), jnp.float32),
            jax.ShapeDtypeStruct((2, 3, N_PAD // 16, 16), jnp.float32),
        ],
        mesh=_MESH,
        compiler_params=pltpu.CompilerParams(needs_layout_passes=False, use_tc_tiling_on_sc=False),
        scratch_types=[
            pltpu.VMEM((NNZ_T,), jnp.int32),
            pltpu.VMEM((NNZ_T,), jnp.int32),
            pltpu.VMEM((NNZ_T,), jnp.float32),
            pltpu.VMEM((N_PAD,), jnp.float32),
            pltpu.VMEM((N_PAD,), jnp.float32),
            pltpu.VMEM((16,), jnp.float32),
            pltpu.VMEM((5, 128), jnp.int32),
            pltpu.VMEM((N_PAD // 16, 16), jnp.float32),
            pltpu.VMEM((N_PAD // 16, 16), jnp.float32),
            pltpu.VMEM((N_PAD // 16, 16), jnp.float32),
            pltpu.VMEM((40, 16), jnp.float32),
            pltpu.VMEM_SHARED((N_PAD // 16, 16), jnp.float32),
            pltpu.VMEM_SHARED((N_PAD // 16, 16), jnp.float32),
            pltpu.VMEM_SHARED((N_PAD // 16, 16), jnp.float32),
        ],
    )
    return kfn(nid2, eid2, a_n, a_e, shift16, riota)


def _mid1_body(p_ref, out_ref):
    p = p_ref[...]
    m = p[0] + p[1]
    es = m[0:1]
    D = m[1:2]
    B = m[2:3]
    esinv = 1.0 / (es + 1e-16)
    dinv = jnp.where(D > 0, 1.0 / D, 0.0)
    binv = jnp.where(B > 0, 1.0 / B, 0.0)
    out_ref[...] = jnp.concatenate([esinv, dinv, binv], axis=0)


def _mid1(part):
    return pl.pallas_call(
        _mid1_body,
        out_shape=jax.ShapeDtypeStruct((3, N_PAD), jnp.float32),
    )(part)


def _sc_weights_body(nid_hbm, eid_hbm, ex_hbm, esi_hbm, dinv_hbm, binv_hbm,
                     w1_hbm, w2_hbm,
                     nid_v, eid_v, ex_v, w1_v, w2_v, esi_t, dinv_t, binv_t):
    c = lax.axis_index("c")
    s = lax.axis_index("s")
    w = c * 16 + s
    pltpu.sync_copy(esi_hbm, esi_t)
    pltpu.sync_copy(dinv_hbm, dinv_t)
    pltpu.sync_copy(binv_hbm, binv_t)
    H = NNZ_T // 2
    for half in range(2):
        hsl = pl.ds(half * H, H)
        pltpu.sync_copy(nid_hbm.at[w, hsl], nid_v)
        pltpu.sync_copy(eid_hbm.at[w, hsl], eid_v)
        pltpu.sync_copy(ex_hbm.at[w, hsl], ex_v)

        def body(i, _):
            sl = pl.ds(i * 16, 16)
            ni = nid_v[sl]
            ei = eid_v[sl]
            base = ex_v[sl] * plsc.load_gather(esi_t, [ni])
            w1_v[sl] = base * plsc.load_gather(binv_t, [ei])
            w2_v[sl] = base * plsc.load_gather(dinv_t, [ni])
            return 0
        lax.fori_loop(0, H // 16, body, 0)
        pltpu.sync_copy(w1_v, w1_hbm.at[w, hsl])
        pltpu.sync_copy(w2_v, w2_hbm.at[w, hsl])


def _sc_weights(nid2, eid2, ex2, esinv, dinv, binv):
    kfn = pl.kernel(
        _sc_weights_body,
        out_type=[
            jax.ShapeDtypeStruct((NW, NNZ_T), jnp.float32),
            jax.ShapeDtypeStruct((NW, NNZ_T), jnp.float32),
        ],
        mesh=_MESH,
        compiler_params=pltpu.CompilerParams(needs_layout_passes=False, use_tc_tiling_on_sc=False),
        scratch_types=(
            [pltpu.VMEM((NNZ_T // 2,), jnp.int32)] * 2
            + [pltpu.VMEM((NNZ_T // 2,), jnp.float32)] * 3
            + [pltpu.VMEM((N_PAD,), jnp.float32)] * 3
        ),
    )
    return kfn(nid2, eid2, ex2, esinv, dinv, binv)


def _sc_spmm_body(src_hbm, gi_hbm, si_hbm, w_hbm,
                  part_hbm,
                  gi_v, si_v, w_v, gb0, gb1, sb0, sb1, gsem0, gsem1, ssem0, ssem1,
                  shacc):
    c = lax.axis_index("c")
    s = lax.axis_index("s")
    w = c * 16 + s
    pltpu.sync_copy(gi_hbm.at[w], gi_v)
    pltpu.sync_copy(si_hbm.at[w], si_v)
    pltpu.sync_copy(w_hbm.at[w], w_v)

    z16 = jnp.zeros((16,), jnp.float32)

    def zr_body(i, _):
        for cc in range(FH // 16):
            sb0[i, pl.ds(cc * 16, 16)] = z16
        return 0
    lax.fori_loop(0, CHUNK_R, zr_body, 0)
    for q in range(N_PAD // 16 // CHUNK_R):
        pltpu.sync_copy(sb0, shacc.at[pl.ds(s * (N_PAD // 16) + q * CHUNK_R, CHUNK_R)])
    plsc.subcore_barrier()

    def gsrc(j):
        return src_hbm.at[gi_v.at[pl.ds(j * CHUNK_R, CHUNK_R)]]

    pltpu.async_copy(gsrc(0), gb0, gsem0)
    pltpu.async_copy(gsrc(1), gb1, gsem1)

    def blk(g, _):
        for p, gb, sb, gsem, ssem in ((0, gb0, sb0, gsem0, ssem0),
                                      (1, gb1, sb1, gsem1, ssem1)):
            j = 2 * g + p
            pltpu.make_async_copy(gsrc(j), gb, gsem).wait()

            @pl.when(g > 0)
            def _():
                pltpu.make_async_copy(sb, shacc.at[si_v.at[j]], ssem).wait()

            def row(r, _):
                ws = plsc.load_gather(w_v, [jnp.full((16,), j * CHUNK_R + r, jnp.int32)])
                for cc in range(FH // 16):
                    sl = pl.ds(cc * 16, 16)
                    sb[r, sl] = gb[r, sl] * ws
                return 0
            lax.fori_loop(0, CHUNK_R, row, 0, unroll=8)

            @pl.when(g < NCHUNK // 2 - 1)
            def _():
                pltpu.async_copy(gsrc(j + 2), gb, gsem)

            pltpu.async_copy(sb, shacc.at[si_v.at[j]], ssem, add=True)
        return 0
    lax.fori_loop(0, NCHUNK // 2, blk, 0)
    pltpu.make_async_copy(sb0, shacc.at[si_v.at[0]], ssem0).wait()
    pltpu.make_async_copy(sb1, shacc.at[si_v.at[0]], ssem1).wait()
    plsc.subcore_barrier()
    rsl = pl.ds(s * 640, 640)
    pltpu.sync_copy(shacc.at[rsl], part_hbm.at[c, rsl])


def _sc_spmm(src_h, gi_flat, si_2d, w_flat):
    kfn = pl.kernel(
        _sc_spmm_body,
        out_type=jax.ShapeDtypeStruct((2, N_PAD, FH), jnp.float32),
        mesh=_MESH,
        compiler_params=pltpu.CompilerParams(needs_layout_passes=False, use_tc_tiling_on_sc=False),
        scratch_types=[
            pltpu.VMEM((NNZ_T,), jnp.int32),
            pltpu.VMEM((NCHUNK, CHUNK_R), jnp.int32),
            pltpu.VMEM((NNZ_T,), jnp.float32),
            pltpu.VMEM((CHUNK_R, FH), jnp.float32),
            pltpu.VMEM((CHUNK_R, FH), jnp.float32),
            pltpu.VMEM((CHUNK_R, FH), jnp.float32),
            pltpu.VMEM((CHUNK_R, FH), jnp.float32),
            pltpu.SemaphoreType.DMA,
            pltpu.SemaphoreType.DMA,
            pltpu.SemaphoreType.DMA,
            pltpu.SemaphoreType.DMA,
            pltpu.VMEM_SHARED((N_PAD, FH), jnp.float32),
        ],
    )
    return kfn(src_h, gi_flat, si_2d, w_flat)


def _merge_body(pa_ref, pb_ref, oa_ref, ob_ref):
    oa_ref[...] = pa_ref[...].sum(axis=0)
    ob_ref[...] = pb_ref[...].sum(axis=0)


def _merge(pa, pb):
    return pl.pallas_call(
        _merge_body,
        grid=(N_PAD // 512,),
        in_specs=[pl.BlockSpec((2, 512, FH), lambda i: (0, i, 0)),
                  pl.BlockSpec((2, 512, FH), lambda i: (0, i, 0))],
        out_specs=[pl.BlockSpec((512, FH), lambda i: (i, 0)),
                   pl.BlockSpec((512, FH), lambda i: (i, 0))],
        out_shape=[jax.ShapeDtypeStruct((N_PAD, FH), jnp.float32),
                   jax.ShapeDtypeStruct((N_PAD, FH), jnp.float32)],
    )(pa, pb)


def _post_body(pa_ref, pb_ref, tvec_ref, out_ref):
    h = jnp.concatenate([pa_ref[...].sum(axis=0), pb_ref[...].sum(axis=0)], axis=-1)
    h = h + tvec_ref[...]
    out_ref[...] = h * (1.0 / (1.0 + jnp.exp(-h)))


def _postamble(pa, pb, tvec, n_out):
    blk = 1000
    return pl.pallas_call(
        _post_body,
        grid=(n_out // blk,),
        in_specs=[
            pl.BlockSpec((2, blk, FH), lambda i: (0, i, 0)),
            pl.BlockSpec((2, blk, FH), lambda i: (0, i, 0)),
            pl.BlockSpec((1, F), lambda i: (0, 0)),
        ],
        out_specs=pl.BlockSpec((blk, F), lambda i: (i, 0)),
        out_shape=jax.ShapeDtypeStruct((n_out, F), jnp.float32),
    )(pa, pb, tvec)


def kernel(x, hyperedge_index, hyperedge_attr, temb, W, att, b, Wt, bt):
    N = x.shape[0]
    M = hyperedge_attr.shape[0]
    nnz = hyperedge_index.shape[1]
    attn = att[0, :, :F]
    atte = att[0, :, F:]
    x_p = jnp.pad(x, ((0, N_PAD - N), (0, 0)))
    ha_p = jnp.pad(hyperedge_attr, ((0, N_PAD - M), (0, 0)))

    xt, he, an2, ae2, tvec, shift = _preamble(x_p, ha_p, W, attn, atte, temb, Wt, bt, b)
    a_n = an2.reshape(N_PAD)
    a_e = ae2.reshape(N_PAD)
    shift16 = jnp.broadcast_to(shift, (16,))

    idx = hyperedge_index.astype(jnp.int32)
    pad = jnp.full((2, NNZ_PAD - nnz), PAD_IDX, jnp.int32)
    idx_p = jnp.concatenate([idx, pad], axis=1)
    nid2 = idx_p[0].reshape(NW, NNZ_T)
    eid2 = idx_p[1].reshape(NW, NNZ_T)
    nid3 = idx_p[0].reshape(NW, NCHUNK, CHUNK_R)
    eid3 = idx_p[1].reshape(NW, NCHUNK, CHUNK_R)
    riota = jnp.arange(N_PAD // 16, dtype=jnp.int32).reshape(5, 128)

    ex2, sums_part = _sc_logits(nid2, eid2, a_n, a_e, shift16, riota)
    tables = _mid1(sums_part.reshape(2, 3, N_PAD))
    esinv, dinv, binv = tables[0], tables[1], tables[2]

    w1, w2 = _sc_weights(nid2, eid2, ex2, esinv, dinv, binv)

    xtA, xtB = xt[:, :FH], xt[:, FH:]
    oeA = _sc_spmm(xtA, nid2, eid3, w1)
    oeB = _sc_spmm(xtB, nid2, eid3, w1)
    out_eA, out_eB = _merge(oeA, oeB)
    onA = _sc_spmm(out_eA, eid2, nid3, w2)
    onB = _sc_spmm(out_eB, eid2, nid3, w2)
    return _postamble(onA, onB, tvec, N)


# speedup vs baseline: 1.1444x; 1.1444x over previous
import functools

import jax
import jax.numpy as jnp
from jax import lax
from jax.experimental import pallas as pl
from jax.experimental.pallas import tpu as pltpu
from jax.experimental.pallas import tpu_sc as plsc

N_PAD = 10240
NNZ_PAD = 327680
NNZ_T = 10240
NCHUNK = 80
CHUNK_R = 128
FH = 64
F = 128
BLK = 128
PAD_IDX = 10016
NW = 32

_MESH = plsc.VectorSubcoreMesh(core_axis_name="c", subcore_axis_name="s",
                               num_cores=2, num_subcores=16)


def _pre_body(x_ref, ha_ref, W_ref, attn_ref, atte_ref, temb_ref, Wt_ref, bt_ref, b_ref,
              xt_ref, he_ref, an_ref, ae_ref, tvec_ref, shift_ref, mx_ref):
    i = pl.program_id(0)
    W = W_ref[...]
    xt = jnp.dot(x_ref[...], W, preferred_element_type=jnp.float32)
    he = jnp.dot(ha_ref[...], W, preferred_element_type=jnp.float32)
    xt_ref[...] = xt
    he_ref[...] = he
    an = jnp.sum(xt * attn_ref[...], axis=1)
    ae = jnp.sum(he * atte_ref[...], axis=1)
    an_ref[...] = an[None, None, :]
    ae_ref[...] = ae[None, None, :]

    @pl.when(i == 0)
    def _init():
        mx_ref[0] = -jnp.inf
        mx_ref[1] = -jnp.inf
        t = jax.nn.silu(temb_ref[...])
        tvec = jnp.dot(t, Wt_ref[...], preferred_element_type=jnp.float32)
        tvec_ref[...] = tvec + bt_ref[...][None, :] + b_ref[...][None, :]

    mx_ref[0] = jnp.maximum(mx_ref[0], jnp.max(an))
    mx_ref[1] = jnp.maximum(mx_ref[1], jnp.max(ae))

    @pl.when(i == pl.num_programs(0) - 1)
    def _fin():
        m = mx_ref[0] + mx_ref[1]
        shift_ref[0] = jnp.where(m > 0, m, 0.2 * m)


def _preamble(x_p, ha_p, W, attn, atte, temb, Wt, bt, b):
    nblk = N_PAD // BLK
    full = lambda i: (0, 0)
    return pl.pallas_call(
        _pre_body,
        grid=(nblk,),
        in_specs=[
            pl.BlockSpec((BLK, F), lambda i: (i, 0)),
            pl.BlockSpec((BLK, F), lambda i: (i, 0)),
            pl.BlockSpec((F, F), full),
            pl.BlockSpec((1, F), full),
            pl.BlockSpec((1, F), full),
            pl.BlockSpec((1, 512), full),
            pl.BlockSpec((512, F), full),
            pl.BlockSpec((F,), lambda i: (0,)),
            pl.BlockSpec((F,), lambda i: (0,)),
        ],
        out_specs=[
            pl.BlockSpec((BLK, F), lambda i: (i, 0)),
            pl.BlockSpec((BLK, F), lambda i: (i, 0)),
            pl.BlockSpec((1, 1, BLK), lambda i: (i, 0, 0)),
            pl.BlockSpec((1, 1, BLK), lambda i: (i, 0, 0)),
            pl.BlockSpec((1, F), full),
            pl.BlockSpec(memory_space=pltpu.SMEM),
        ],
        out_shape=[
            jax.ShapeDtypeStruct((N_PAD, F), jnp.float32),
            jax.ShapeDtypeStruct((N_PAD, F), jnp.float32),
            jax.ShapeDtypeStruct((nblk, 1, BLK), jnp.float32),
            jax.ShapeDtypeStruct((nblk, 1, BLK), jnp.float32),
            jax.ShapeDtypeStruct((1, F), jnp.float32),
            jax.ShapeDtypeStruct((1,), jnp.float32),
        ],
        scratch_shapes=[pltpu.SMEM((2,), jnp.float32)],
    )(x_p, ha_p, W, attn, atte, temb, Wt, bt, b)


def _sc_logits_body(nid_hbm, eid_hbm, an_hbm, ae_hbm, shift_hbm, riota_hbm,
                    ex_hbm, part_hbm,
                    nid_v, eid_v, ex_v, an_t, ae_t, sh16, riota_v,
                    acc_es, acc_d, acc_b, zbuf,
                    sh_es, sh_d, sh_b):
    c = lax.axis_index("c")
    s = lax.axis_index("s")
    w = c * 16 + s
    pltpu.sync_copy(an_hbm, an_t)
    pltpu.sync_copy(ae_hbm, ae_t)
    pltpu.sync_copy(shift_hbm, sh16)
    pltpu.sync_copy(riota_hbm, riota_v)
    pltpu.sync_copy(nid_hbm.at[w], nid_v)
    pltpu.sync_copy(eid_hbm.at[w], eid_v)
    shv = sh16[...]
    z16 = jnp.zeros((16,), jnp.float32)

    def zero_body(i, _):
        acc_es[i, :] = z16
        acc_d[i, :] = z16
        acc_b[i, :] = z16
        return 0
    lax.fori_loop(0, N_PAD // 16, zero_body, 0)

    def zb_body(i, _):
        zbuf[i, :] = z16
        return 0
    lax.fori_loop(0, 40, zb_body, 0)

    ones16 = jnp.ones((16,), jnp.float32)

    def body(i, _):
        sl = pl.ds(i * 16, 16)
        ni = nid_v[sl]
        ei = eid_v[sl]
        a = plsc.load_gather(an_t, [ni]) + plsc.load_gather(ae_t, [ei])
        a = jnp.where(a > 0, a, 0.2 * a)
        ex = jnp.exp(a - shv)
        ex_v[sl] = ex
        nhi, nlo = ni // 16, ni % 16
        plsc.addupdate_scatter(acc_es, [nhi, nlo], ex)
        plsc.addupdate_scatter(acc_d, [nhi, nlo], ones16)
        plsc.addupdate_scatter(acc_b, [ei // 16, ei % 16], ones16)
        return 0
    lax.fori_loop(0, NNZ_T // 16, body, 0)
    pltpu.sync_copy(ex_v, ex_hbm.at[w])

    rsl = pl.ds(s * 40, 40)
    pltpu.sync_copy(zbuf, sh_es.at[rsl])
    pltpu.sync_copy(zbuf, sh_d.at[rsl])
    pltpu.sync_copy(zbuf, sh_b.at[rsl])
    plsc.subcore_barrier()
    for j in range(5):
        src = pl.ds(j * 128, 128)
        idx = riota_v.at[j]
        pltpu.sync_copy(acc_es.at[src], sh_es.at[idx], add=True)
        pltpu.sync_copy(acc_d.at[src], sh_d.at[idx], add=True)
        pltpu.sync_copy(acc_b.at[src], sh_b.at[idx], add=True)
    plsc.subcore_barrier()
    pltpu.sync_copy(sh_es.at[rsl], part_hbm.at[c, 0, rsl])
    pltpu.sync_copy(sh_d.at[rsl], part_hbm.at[c, 1, rsl])
    pltpu.sync_copy(sh_b.at[rsl], part_hbm.at[c, 2, rsl])


def _sc_logits(nid2, eid2, a_n, a_e, shift16, riota):
    kfn = pl.kernel(
        _sc_logits_body,
        out_type=[
            jax.ShapeDtypeStruct((NW, NNZ_T), jnp.float32),
            jax.ShapeDtypeStruct((2, 3, N_PAD // 16, 16), jnp.float32),
        ],
        mesh=_MESH,
        compiler_params=pltpu.CompilerParams(needs_layout_passes=False, use_tc_tiling_on_sc=False),
        scratch_types=[
            pltpu.VMEM((NNZ_T,), jnp.int32),
            pltpu.VMEM((NNZ_T,), jnp.int32),
            pltpu.VMEM((NNZ_T,), jnp.float32),
            pltpu.VMEM((N_PAD,), jnp.float32),
            pltpu.VMEM((N_PAD,), jnp.float32),
            pltpu.VMEM((16,), jnp.float32),
            pltpu.VMEM((5, 128), jnp.int32),
            pltpu.VMEM((N_PAD // 16, 16), jnp.float32),
            pltpu.VMEM((N_PAD // 16, 16), jnp.float32),
            pltpu.VMEM((N_PAD // 16, 16), jnp.float32),
            pltpu.VMEM((40, 16), jnp.float32),
            pltpu.VMEM_SHARED((N_PAD // 16, 16), jnp.float32),
            pltpu.VMEM_SHARED((N_PAD // 16, 16), jnp.float32),
            pltpu.VMEM_SHARED((N_PAD // 16, 16), jnp.float32),
        ],
    )
    return kfn(nid2, eid2, a_n, a_e, shift16, riota)


def _mid1_body(p_ref, out_ref):
    p = p_ref[...]
    m = p[0] + p[1]
    es = m[0:1]
    D = m[1:2]
    B = m[2:3]
    esinv = 1.0 / (es + 1e-16)
    dinv = jnp.where(D > 0, 1.0 / D, 0.0)
    binv = jnp.where(B > 0, 1.0 / B, 0.0)
    out_ref[...] = jnp.concatenate([esinv, dinv, binv], axis=0)


def _mid1(part):
    return pl.pallas_call(
        _mid1_body,
        out_shape=jax.ShapeDtypeStruct((3, N_PAD), jnp.float32),
    )(part)


def _sc_weights_body(nid_hbm, eid_hbm, ex_hbm, esi_hbm, dinv_hbm, binv_hbm,
                     w1_hbm, w2_hbm,
                     nid_v, eid_v, ex_v, w1_v, w2_v, esi_t, dinv_t, binv_t):
    c = lax.axis_index("c")
    s = lax.axis_index("s")
    w = c * 16 + s
    pltpu.sync_copy(esi_hbm, esi_t)
    pltpu.sync_copy(dinv_hbm, dinv_t)
    pltpu.sync_copy(binv_hbm, binv_t)
    H = NNZ_T // 2
    for half in range(2):
        hsl = pl.ds(half * H, H)
        pltpu.sync_copy(nid_hbm.at[w, hsl], nid_v)
        pltpu.sync_copy(eid_hbm.at[w, hsl], eid_v)
        pltpu.sync_copy(ex_hbm.at[w, hsl], ex_v)

        def body(i, _):
            sl = pl.ds(i * 16, 16)
            ni = nid_v[sl]
            ei = eid_v[sl]
            base = ex_v[sl] * plsc.load_gather(esi_t, [ni])
            w1_v[sl] = base * plsc.load_gather(binv_t, [ei])
            w2_v[sl] = base * plsc.load_gather(dinv_t, [ni])
            return 0
        lax.fori_loop(0, H // 16, body, 0)
        pltpu.sync_copy(w1_v, w1_hbm.at[w, hsl])
        pltpu.sync_copy(w2_v, w2_hbm.at[w, hsl])


def _sc_weights(nid2, eid2, ex2, esinv, dinv, binv):
    kfn = pl.kernel(
        _sc_weights_body,
        out_type=[
            jax.ShapeDtypeStruct((NW, NNZ_T), jnp.float32),
            jax.ShapeDtypeStruct((NW, NNZ_T), jnp.float32),
        ],
        mesh=_MESH,
        compiler_params=pltpu.CompilerParams(needs_layout_passes=False, use_tc_tiling_on_sc=False),
        scratch_types=(
            [pltpu.VMEM((NNZ_T // 2,), jnp.int32)] * 2
            + [pltpu.VMEM((NNZ_T // 2,), jnp.float32)] * 3
            + [pltpu.VMEM((N_PAD,), jnp.float32)] * 3
        ),
    )
    return kfn(nid2, eid2, ex2, esinv, dinv, binv)


def _sc_spmm_body(src_hbm, gi_hbm, si_hbm, w_hbm,
                  part_hbm,
                  gi_v, si_v, w_v, gb0, gb1, sb0, sb1, gsem0, gsem1, ssem0, ssem1,
                  shacc):
    c = lax.axis_index("c")
    s = lax.axis_index("s")
    w = c * 16 + s
    pltpu.sync_copy(gi_hbm.at[w], gi_v)
    pltpu.sync_copy(si_hbm.at[w], si_v)
    pltpu.sync_copy(w_hbm.at[w], w_v)

    z16 = jnp.zeros((16,), jnp.float32)

    def zr_body(i, _):
        for cc in range(FH // 16):
            sb0[i, pl.ds(cc * 16, 16)] = z16
        return 0
    lax.fori_loop(0, CHUNK_R, zr_body, 0)
    for q in range(N_PAD // 16 // CHUNK_R):
        pltpu.sync_copy(sb0, shacc.at[pl.ds(s * (N_PAD // 16) + q * CHUNK_R, CHUNK_R)])
    plsc.subcore_barrier()

    def gsrc(j):
        return src_hbm.at[gi_v.at[pl.ds(j * CHUNK_R, CHUNK_R)]]

    pltpu.async_copy(gsrc(0), gb0, gsem0)
    pltpu.async_copy(gsrc(1), gb1, gsem1)

    def blk(g, _):
        for p, gb, sb, gsem, ssem in ((0, gb0, sb0, gsem0, ssem0),
                                      (1, gb1, sb1, gsem1, ssem1)):
            j = 2 * g + p
            pltpu.make_async_copy(gsrc(j), gb, gsem).wait()

            @pl.when(g > 0)
            def _():
                pltpu.make_async_copy(sb, shacc.at[si_v.at[j]], ssem).wait()

            def row16(g16, _):
                base = j * CHUNK_R + g16 * 16
                w16 = w_v[pl.ds(base, 16)]
                for r in range(16):
                    wsp = lax.gather(
                        w16, jnp.full((16, 1), r, jnp.int32),
                        lax.GatherDimensionNumbers(offset_dims=(),
                                                   collapsed_slice_dims=(0,),
                                                   start_index_map=(0,)),
                        (1,), mode=lax.GatherScatterMode.PROMISE_IN_BOUNDS)
                    rr = g16 * 16 + r
                    for cc in range(FH // 16):
                        sl = pl.ds(cc * 16, 16)
                        sb[rr, sl] = gb[rr, sl] * wsp
                return 0
            lax.fori_loop(0, CHUNK_R // 16, row16, 0, unroll=2)

            @pl.when(g < NCHUNK // 2 - 1)
            def _():
                pltpu.async_copy(gsrc(j + 2), gb, gsem)

            pltpu.async_copy(sb, shacc.at[si_v.at[j]], ssem, add=True)
        return 0
    lax.fori_loop(0, NCHUNK // 2, blk, 0)
    pltpu.make_async_copy(sb0, shacc.at[si_v.at[0]], ssem0).wait()
    pltpu.make_async_copy(sb1, shacc.at[si_v.at[0]], ssem1).wait()
    plsc.subcore_barrier()
    rsl = pl.ds(s * 640, 640)
    pltpu.sync_copy(shacc.at[rsl], part_hbm.at[c, rsl])


def _sc_spmm(src_h, gi_flat, si_2d, w_flat):
    kfn = pl.kernel(
        _sc_spmm_body,
        out_type=jax.ShapeDtypeStruct((2, N_PAD, FH), jnp.float32),
        mesh=_MESH,
        compiler_params=pltpu.CompilerParams(needs_layout_passes=False, use_tc_tiling_on_sc=False),
        scratch_types=[
            pltpu.VMEM((NNZ_T,), jnp.int32),
            pltpu.VMEM((NCHUNK, CHUNK_R), jnp.int32),
            pltpu.VMEM((NNZ_T,), jnp.float32),
            pltpu.VMEM((CHUNK_R, FH), jnp.float32),
            pltpu.VMEM((CHUNK_R, FH), jnp.float32),
            pltpu.VMEM((CHUNK_R, FH), jnp.float32),
            pltpu.VMEM((CHUNK_R, FH), jnp.float32),
            pltpu.SemaphoreType.DMA,
            pltpu.SemaphoreType.DMA,
            pltpu.SemaphoreType.DMA,
            pltpu.SemaphoreType.DMA,
            pltpu.VMEM_SHARED((N_PAD, FH), jnp.float32),
        ],
    )
    return kfn(src_h, gi_flat, si_2d, w_flat)


def _merge_body(pa_ref, pb_ref, oa_ref, ob_ref):
    oa_ref[...] = pa_ref[...].sum(axis=0)
    ob_ref[...] = pb_ref[...].sum(axis=0)


def _merge(pa, pb):
    return pl.pallas_call(
        _merge_body,
        grid=(N_PAD // 512,),
        in_specs=[pl.BlockSpec((2, 512, FH), lambda i: (0, i, 0)),
                  pl.BlockSpec((2, 512, FH), lambda i: (0, i, 0))],
        out_specs=[pl.BlockSpec((512, FH), lambda i: (i, 0)),
                   pl.BlockSpec((512, FH), lambda i: (i, 0))],
        out_shape=[jax.ShapeDtypeStruct((N_PAD, FH), jnp.float32),
                   jax.ShapeDtypeStruct((N_PAD, FH), jnp.float32)],
    )(pa, pb)


def _post_body(pa_ref, pb_ref, tvec_ref, out_ref):
    h = jnp.concatenate([pa_ref[...].sum(axis=0), pb_ref[...].sum(axis=0)], axis=-1)
    h = h + tvec_ref[...]
    out_ref[...] = h * (1.0 / (1.0 + jnp.exp(-h)))


def _postamble(pa, pb, tvec, n_out):
    blk = 1000
    return pl.pallas_call(
        _post_body,
        grid=(n_out // blk,),
        in_specs=[
            pl.BlockSpec((2, blk, FH), lambda i: (0, i, 0)),
            pl.BlockSpec((2, blk, FH), lambda i: (0, i, 0)),
            pl.BlockSpec((1, F), lambda i: (0, 0)),
        ],
        out_specs=pl.BlockSpec((blk, F), lambda i: (i, 0)),
        out_shape=jax.ShapeDtypeStruct((n_out, F), jnp.float32),
    )(pa, pb, tvec)


def kernel(x, hyperedge_index, hyperedge_attr, temb, W, att, b, Wt, bt):
    N = x.shape[0]
    M = hyperedge_attr.shape[0]
    nnz = hyperedge_index.shape[1]
    attn = att[0, :, :F]
    atte = att[0, :, F:]
    x_p = jnp.pad(x, ((0, N_PAD - N), (0, 0)))
    ha_p = jnp.pad(hyperedge_attr, ((0, N_PAD - M), (0, 0)))

    xt, he, an2, ae2, tvec, shift = _preamble(x_p, ha_p, W, attn, atte, temb, Wt, bt, b)
    a_n = an2.reshape(N_PAD)
    a_e = ae2.reshape(N_PAD)
    shift16 = jnp.broadcast_to(shift, (16,))

    idx = hyperedge_index.astype(jnp.int32)
    pad = jnp.full((2, NNZ_PAD - nnz), PAD_IDX, jnp.int32)
    idx_p = jnp.concatenate([idx, pad], axis=1)
    nid2 = idx_p[0].reshape(NW, NNZ_T)
    eid2 = idx_p[1].reshape(NW, NNZ_T)
    nid3 = idx_p[0].reshape(NW, NCHUNK, CHUNK_R)
    eid3 = idx_p[1].reshape(NW, NCHUNK, CHUNK_R)
    riota = jnp.arange(N_PAD // 16, dtype=jnp.int32).reshape(5, 128)

    ex2, sums_part = _sc_logits(nid2, eid2, a_n, a_e, shift16, riota)
    tables = _mid1(sums_part.reshape(2, 3, N_PAD))
    esinv, dinv, binv = tables[0], tables[1], tables[2]

    w1, w2 = _sc_weights(nid2, eid2, ex2, esinv, dinv, binv)

    xtA, xtB = xt[:, :FH], xt[:, FH:]
    oeA = _sc_spmm(xtA, nid2, eid3, w1)
    oeB = _sc_spmm(xtB, nid2, eid3, w1)
    out_eA, out_eB = _merge(oeA, oeB)
    onA = _sc_spmm(out_eA, eid2, nid3, w2)
    onB = _sc_spmm(out_eB, eid2, nid3, w2)
    return _postamble(onA, onB, tvec, N)
